# baseline (speedup 1.0000x reference)
import functools

import jax
import jax.numpy as jnp
from jax import lax
from jax.experimental import pallas as pl
from jax.experimental.pallas import tpu as pltpu
from jax.experimental.pallas import tpu_sc as plsc

NC, NS, L = 2, 16, 16
NW = NC * NS
NBUF = 4


@functools.partial(jax.jit, static_argnums=(4, 5))
def _lookup(table, idx_t, oidx, pos, spw, d):
    t_len = pos.shape[0]
    n_rows = NW * t_len * spw

    mesh = plsc.VectorSubcoreMesh(core_axis_name="c", subcore_axis_name="s")

    @functools.partial(
        pl.kernel,
        mesh=mesh,
        out_type=jax.ShapeDtypeStruct((n_rows, d), jnp.float32),
        scratch_types=[
            pltpu.VMEM((t_len, spw), jnp.int32),
            pltpu.VMEM((t_len, spw), jnp.int32),
        ]
        + [pltpu.VMEM((spw, d), jnp.float32) for _ in range(NBUF)]
        + [pltpu.VMEM((1, d), jnp.float32) for _ in range(NBUF)]
        + [pltpu.SemaphoreType.DMA for _ in range(3 * NBUF)],
    )
    def body(table_hbm, idx_hbm, oidx_hbm, pos_hbm, out_hbm, idx_v, oidx_v, *rest):
        bufs = rest[:NBUF]
        pbufs = rest[NBUF:2 * NBUF]
        sin = rest[2 * NBUF:3 * NBUF]
        sout = rest[3 * NBUF:4 * NBUF]
        spos = rest[4 * NBUF:]

        wid = lax.axis_index("s") * NC + lax.axis_index("c")
        pltpu.sync_copy(idx_hbm.at[wid], idx_v)
        pltpu.sync_copy(oidx_hbm.at[wid], oidx_v)

        def gather_start(k, b):
            pltpu.async_copy(table_hbm.at[idx_v.at[k]], bufs[b], sin[b])
            pltpu.async_copy(pos_hbm.at[pl.ds(k, 1)], pbufs[b], spos[b])

        def gather_wait(k, b):
            pltpu.make_async_copy(table_hbm.at[idx_v.at[k]], bufs[b], sin[b]).wait()
            pltpu.make_async_copy(pos_hbm.at[pl.ds(k, 1)], pbufs[b], spos[b]).wait()

        def scatter_start(k, b):
            pltpu.async_copy(bufs[b], out_hbm.at[oidx_v.at[k]], sout[b])

        def scatter_wait(k, b):
            pltpu.make_async_copy(bufs[b], out_hbm.at[oidx_v.at[k]], sout[b]).wait()

        def compute(b):
            buf = bufs[b]
            pbuf = pbufs[b]

            def jbody(j, carry):
                col = j * L
                pvec = pbuf[0, pl.ds(col, L)]
                for r in range(spw):
                    buf[r, pl.ds(col, L)] = buf[r, pl.ds(col, L)] + pvec
                return carry

            lax.fori_loop(0, d // L, jbody, 0)

        gather_start(0, 0)
        gather_start(1, 1)

        def kbody(k, carry):
            bsel = lax.rem(k, NBUF)
            for b in range(NBUF):
                bp = (b + 2) % NBUF

                @pl.when(bsel == b)
                def _():
                    @pl.when(k + 2 < t_len)
                    def _():
                        @pl.when(k >= 2)
                        def _():
                            scatter_wait(k - 2, bp)

                        gather_start(k + 2, bp)

                    gather_wait(k, b)
                    compute(b)
                    scatter_start(k, b)
            return carry

        lax.fori_loop(0, t_len, kbody, 0)

        for k in range(t_len - NBUF, t_len):
            scatter_wait(k, k % NBUF)

    return body(table, idx_t, oidx, pos)


def kernel(tokens, token_embedding, position_embedding):
    bsz, t_len = tokens.shape
    _, d = token_embedding.shape
    spw = bsz // NW
    idx_t = jnp.transpose(
        tokens.astype(jnp.int32).reshape(NW, spw, t_len), (0, 2, 1)
    )
    rows = (
        (jnp.arange(NW, dtype=jnp.int32) * spw)[:, None, None]
        + jnp.arange(spw, dtype=jnp.int32)[None, None, :]
    ) * t_len + jnp.arange(t_len, dtype=jnp.int32)[None, :, None]
    out = _lookup(token_embedding, idx_t, rows, position_embedding, spw, d)
    return out.reshape(bsz, t_len, d)


# speedup vs baseline: 2.7205x; 2.7205x over previous
import functools

import jax
import jax.numpy as jnp
from jax import lax
from jax.experimental import pallas as pl
from jax.experimental.pallas import tpu as pltpu
from jax.experimental.pallas import tpu_sc as plsc

NC, NS, L = 2, 16, 16
NW = NC * NS
NBUF = 4


@functools.partial(jax.jit, static_argnums=(4, 5))
def _lookup(table, idx_t, oidx, pos, spw, d):
    t_len = pos.shape[0]
    n_rows = NW * t_len * spw

    mesh = plsc.VectorSubcoreMesh(core_axis_name="c", subcore_axis_name="s")

    @functools.partial(
        pl.kernel,
        mesh=mesh,
        out_type=jax.ShapeDtypeStruct((n_rows, d), jnp.float32),
        scratch_types=[
            pltpu.VMEM((t_len, spw), jnp.int32),
            pltpu.VMEM((t_len, spw), jnp.int32),
        ]
        + [pltpu.VMEM((spw, d), jnp.float32) for _ in range(NBUF)]
        + [pltpu.VMEM((1, d), jnp.float32) for _ in range(NBUF)]
        + [pltpu.SemaphoreType.DMA for _ in range(3 * NBUF)],
    )
    def body(table_hbm, idx_hbm, oidx_hbm, pos_hbm, out_hbm, idx_v, oidx_v, *rest):
        bufs = rest[:NBUF]
        pbufs = rest[NBUF:2 * NBUF]
        sin = rest[2 * NBUF:3 * NBUF]
        sout = rest[3 * NBUF:4 * NBUF]
        spos = rest[4 * NBUF:]

        wid = lax.axis_index("s") * NC + lax.axis_index("c")
        pltpu.sync_copy(idx_hbm.at[wid], idx_v)
        pltpu.sync_copy(oidx_hbm.at[wid], oidx_v)

        def gather_start(k, b):
            pltpu.async_copy(table_hbm.at[idx_v.at[k]], bufs[b], sin[b])
            pltpu.async_copy(pos_hbm.at[pl.ds(k, 1)], pbufs[b], spos[b])

        def gather_wait(k, b):
            pltpu.make_async_copy(table_hbm.at[idx_v.at[k]], bufs[b], sin[b]).wait()
            pltpu.make_async_copy(pos_hbm.at[pl.ds(k, 1)], pbufs[b], spos[b]).wait()

        def scatter_start(k, b):
            pltpu.async_copy(bufs[b], out_hbm.at[oidx_v.at[k]], sout[b])

        def scatter_wait(k, b):
            pltpu.make_async_copy(bufs[b], out_hbm.at[oidx_v.at[k]], sout[b]).wait()

        def compute(b):
            buf = bufs[b]
            pbuf = pbufs[b]

            def jbody(j, carry):
                col = j * L
                pvec = pbuf[0, pl.ds(col, L)]
                for r in range(spw):
                    buf[r, pl.ds(col, L)] = buf[r, pl.ds(col, L)] + pvec
                return carry

            lax.fori_loop(0, d // L, jbody, 0)

        gather_start(0, 0)
        gather_start(1, 1)

        def kbody(k, carry):
            bsel = lax.rem(k, NBUF)
            for b in range(NBUF):
                bp = (b + 2) % NBUF

                @pl.when(bsel == b)
                def _():
                    @pl.when(k + 2 < t_len)
                    def _():
                        @pl.when(k >= 2)
                        def _():
                            scatter_wait(k - 2, bp)

                        gather_start(k + 2, bp)

                    gather_wait(k, b)
                    compute(b)
                    scatter_start(k, b)
            return carry

        lax.fori_loop(0, t_len, kbody, 0)

        for k in range(t_len - NBUF, t_len):
            scatter_wait(k, k % NBUF)

    return body(table, idx_t, oidx, pos)


def kernel(tokens, token_embedding, position_embedding):
    bsz, t_len = tokens.shape
    _, d = token_embedding.shape
    spw = bsz // NW
    idx_t = jnp.transpose(
        tokens.astype(jnp.int32).reshape(NW, spw, t_len), (0, 2, 1)
    )
    rows = (
        (jnp.arange(NW, dtype=jnp.int32) * spw)[:, None, None]
        + jnp.arange(spw, dtype=jnp.int32)[None, None, :]
    ) * t_len + jnp.arange(t_len, dtype=jnp.int32)[None, :, None]
    out = _lookup(token_embedding, idx_t, rows, position_embedding, spw, d)
    return out
